# Initial kernel scaffold; baseline (speedup 1.0000x reference)
import jax
import jax.numpy as jnp
from jax.experimental import pallas as pl


def kernel(token_ids, idf_weights):
    raise NotImplementedError("write your pallas kernel here")


# speedup vs baseline: 22.8445x; 22.8445x over previous
import functools

import jax
import jax.numpy as jnp
from jax import lax
from jax.experimental import pallas as pl
from jax.experimental.pallas import tpu as pltpu
from jax.experimental.pallas import tpu_sc as plsc

_NC = 2
_NS = 16
_LANES = 16
_NW = _NC * _NS


def kernel(token_ids, idf_weights):
    B, L = token_ids.shape
    V = idf_weights.shape[0]

    rows_per_w = B // _NW
    groups = rows_per_w // _LANES
    n_full = V // _LANES
    tail_off = V - _LANES

    mesh = plsc.VectorSubcoreMesh(core_axis_name="c", subcore_axis_name="s")

    @functools.partial(
        pl.kernel,
        out_type=jax.ShapeDtypeStruct((B, V), jnp.float32),
        mesh=mesh,
        compiler_params=pltpu.CompilerParams(
            use_tc_tiling_on_sc=False, needs_layout_passes=False),
        scratch_types=[
            pltpu.VMEM((_LANES, L), jnp.int32),
            pltpu.VMEM((V,), jnp.float32),
            pltpu.VMEM((_LANES, V), jnp.float32),
        ],
    )
    def _tfidf(tok_hbm, idf_hbm, out_hbm, tok_v, idf_v, acc_v):
        wid = lax.axis_index("s") * _NC + lax.axis_index("c")
        base = wid * rows_per_w
        pltpu.sync_copy(idf_hbm, idf_v)
        lanes = lax.iota(jnp.int32, _LANES)
        zeros = jnp.zeros((_LANES,), jnp.float32)

        def group_body(g, carry):
            row0 = base + g * _LANES
            pltpu.sync_copy(tok_hbm.at[pl.ds(row0, _LANES), :], tok_v)

            def zero_body(c, carry2):
                off = pl.multiple_of(c * _LANES, _LANES)
                for l in range(_LANES):
                    acc_v[l, pl.ds(off, _LANES)] = zeros
                return carry2

            lax.fori_loop(0, n_full, zero_body, 0, unroll=False)
            for l in range(_LANES):
                acc_v[l, pl.ds(tail_off, _LANES)] = zeros

            def tok_body(j, carry2):
                jv = jnp.full((_LANES,), j, jnp.int32)
                tok = plsc.load_gather(tok_v, [lanes, jv])
                val = plsc.load_gather(idf_v, [tok])
                plsc.addupdate_scatter(acc_v, [lanes, tok], val)
                return carry2

            lax.fori_loop(0, L, tok_body, 0, unroll=False)

            pltpu.sync_copy(acc_v, out_hbm.at[pl.ds(row0, _LANES), :])
            return carry

        lax.fori_loop(0, groups, group_body, 0, unroll=False)

    return _tfidf(token_ids, idf_weights)
